# Initial kernel scaffold; baseline (speedup 1.0000x reference)
import jax
import jax.numpy as jnp
from jax.experimental import pallas as pl


def kernel(cloth_properties, external_forces, gaussian_positions, gaussian_scales, gaussian_rotations, gaussian_opacities, gaussian_features, num_steps):
    raise NotImplementedError("write your pallas kernel here")


# speedup vs baseline: 1.0497x; 1.0497x over previous
import functools

import jax
import jax.numpy as jnp
from jax import lax
from jax.experimental import pallas as pl
from jax.experimental.pallas import tpu as pltpu
from jax.experimental.pallas import tpu_sc as plsc

N = 10000
NC, NS = 2, 16
W = NC * NS
C = 320
H = 16
HW = C + 2 * H
PL2 = W * C + 2 * H

DT = 0.016
REST = 0.05
GRAV_Y = -9.81
MAGIC = 0x5F3759DF


def _rsqrt(s):
    i = lax.bitcast_convert_type(s, jnp.int32)
    i = MAGIC - lax.shift_right_logical(i, 1)
    r = lax.bitcast_convert_type(i, jnp.float32)
    hs = 0.5 * s
    for _ in range(3):
        r = r * (1.5 - hs * r * r)
    return r


def _step_body(pos_h, vel_h, st_h, npos_h, nvel_h, P, V, ST, F, NP, NV):
    wid = lax.axis_index("s") * NC + lax.axis_index("c")
    _worker(wid, pos_h, vel_h, st_h, npos_h, nvel_h, P, V, ST, F, NP, NV)


def _worker(wid, pos_h, vel_h, st_h, npos_h, nvel_h, P, V, ST, F, NP, NV):
    off = wid * C
    pltpu.sync_copy(pos_h.at[:, pl.ds(off, HW)], P)
    pltpu.sync_copy(vel_h.at[:, pl.ds(off, HW)], V)
    pltpu.sync_copy(st_h.at[:, pl.ds(off, HW)], ST)

    iota = lax.broadcasted_iota(jnp.int32, (16,), 0)
    zero = jnp.zeros((16,), jnp.float32)

    def zero_f(k, c):
        l = 16 * k
        F[0, pl.ds(l, 16)] = zero
        F[1, pl.ds(l, 16)] = zero
        F[2, pl.ds(l, 16)] = zero
        return c

    lax.fori_loop(0, HW // 16, zero_f, 0, unroll=False)

    def pass1(k, c):
        l = 16 * k
        gv = (off - H + l) + iota
        p0x = P[0, pl.ds(l, 16)]
        p0y = P[1, pl.ds(l, 16)]
        p0z = P[2, pl.ds(l, 16)]
        st = ST[0, pl.ds(l, 16)]
        ge0 = gv >= 0
        ax = zero
        ay = zero
        az = zero
        for d in range(1, 10):
            pdx = P[0, pl.ds(l + d, 16)]
            pdy = P[1, pl.ds(l + d, 16)]
            pdz = P[2, pl.ds(l + d, 16)]
            dx = pdx - p0x
            dy = pdy - p0y
            dz = pdz - p0z
            s = dx * dx + dy * dy + dz * dz
            r = _rsqrt(s)
            dist = s * r
            coef = st * (dist - REST) * r
            valid = ge0 & (gv < (N - d)) & (s > 0.0)
            sfx = jnp.where(valid, coef * dx, 0.0)
            sfy = jnp.where(valid, coef * dy, 0.0)
            sfz = jnp.where(valid, coef * dz, 0.0)
            ax = ax + sfx
            ay = ay + sfy
            az = az + sfz
            F[0, pl.ds(l + d, 16)] = F[0, pl.ds(l + d, 16)] - sfx
            F[1, pl.ds(l + d, 16)] = F[1, pl.ds(l + d, 16)] - sfy
            F[2, pl.ds(l + d, 16)] = F[2, pl.ds(l + d, 16)] - sfz
        F[0, pl.ds(l, 16)] = F[0, pl.ds(l, 16)] + ax
        F[1, pl.ds(l, 16)] = F[1, pl.ds(l, 16)] + ay
        F[2, pl.ds(l, 16)] = F[2, pl.ds(l, 16)] + az
        return c

    lax.fori_loop(0, (HW - H) // 16, pass1, 0, unroll=False)

    def pass2(k, c):
        l = H + 16 * k
        o = 16 * k
        gv = (off + o) + iota
        fx = F[0, pl.ds(l, 16)] + ST[3, pl.ds(l, 16)]
        fy = F[1, pl.ds(l, 16)] + ST[4, pl.ds(l, 16)] + GRAV_Y
        fz = F[2, pl.ds(l, 16)] + ST[5, pl.ds(l, 16)]
        px = P[0, pl.ds(l, 16)]
        py = P[1, pl.ds(l, 16)]
        pz = P[2, pl.ds(l, 16)]
        fy = fy + jnp.where(py < -1.0, 1000.0 * (-1.0 - py), 0.0)
        inv = 1.0 / (ST[1, pl.ds(l, 16)] + 1e-8)
        axv = fx * inv
        ayv = fy * inv
        azv = fz * inv
        vx = V[0, pl.ds(l, 16)]
        vy = V[1, pl.ds(l, 16)]
        vz = V[2, pl.ds(l, 16)]
        hdt2 = 0.5 * DT * DT
        npx = px + vx * DT + axv * hdt2
        npy = py + vy * DT + ayv * hdt2
        npz = pz + vz * DT + azv * hdt2
        dampf = 1.0 - ST[2, pl.ds(l, 16)] * DT
        nvx = (vx + axv * DT) * dampf
        nvy = (vy + ayv * DT) * dampf
        nvz = (vz + azv * DT) * dampf
        ok = gv < N
        NP[0, pl.ds(o, 16)] = jnp.where(ok, npx, 0.0)
        NP[1, pl.ds(o, 16)] = jnp.where(ok, npy, 0.0)
        NP[2, pl.ds(o, 16)] = jnp.where(ok, npz, 0.0)
        NV[0, pl.ds(o, 16)] = jnp.where(ok, nvx, 0.0)
        NV[1, pl.ds(o, 16)] = jnp.where(ok, nvy, 0.0)
        NV[2, pl.ds(o, 16)] = jnp.where(ok, nvz, 0.0)
        return c

    lax.fori_loop(0, C // 16, pass2, 0, unroll=False)

    pltpu.sync_copy(NP, npos_h.at[:, pl.ds(H + wid * C, C)])
    pltpu.sync_copy(NV, nvel_h.at[:, pl.ds(H + wid * C, C)])


@functools.cache
def _get_step():
    return functools.partial(
        pl.kernel,
        out_type=(
            jax.ShapeDtypeStruct((3, PL2), jnp.float32),
            jax.ShapeDtypeStruct((3, PL2), jnp.float32),
        ),
        mesh=plsc.VectorSubcoreMesh(core_axis_name="c", subcore_axis_name="s",
                                    num_cores=NC, num_subcores=NS),
        scratch_types=[
            pltpu.VMEM((3, HW), jnp.float32),
            pltpu.VMEM((3, HW), jnp.float32),
            pltpu.VMEM((6, HW), jnp.float32),
            pltpu.VMEM((3, HW), jnp.float32),
            pltpu.VMEM((3, C), jnp.float32),
            pltpu.VMEM((3, C), jnp.float32),
        ],
        compiler_params=pltpu.CompilerParams(use_tc_tiling_on_sc=False),
    )(_step_body)


def kernel(cloth_properties, external_forces, gaussian_positions,
           gaussian_scales, gaussian_rotations, gaussian_opacities,
           gaussian_features, num_steps):
    n = gaussian_positions.shape[0]
    pad = ((0, 0), (H, PL2 - H - n))
    pos0 = jnp.pad(gaussian_positions.T, pad)
    vel0 = jnp.zeros((3, PL2), jnp.float32)
    stat = jnp.concatenate(
        [cloth_properties[:, 0][None],
         cloth_properties[:, 6][None],
         cloth_properties[:, 1][None],
         external_forces.T], axis=0)
    stat = jnp.pad(stat, pad)

    step = _get_step()

    def body(_, carry):
        p, v = carry
        return tuple(step(p, v, stat))

    pos, vel = lax.fori_loop(0, num_steps, body, (pos0, vel0))
    return (pos[:, H:H + n].T, vel[:, H:H + n].T, gaussian_scales,
            gaussian_rotations, gaussian_opacities, gaussian_features)
